# baseline (device time: 15555 ns/iter reference)
import jax
import jax.numpy as jnp
from jax import lax
from jax.experimental import pallas as pl
from jax.experimental.pallas import tpu as pltpu

N_DEV = 4
PEER_ORDER = (1, 3, 2)


def kernel(x, router_W, route_idx, expert_W):
    n_tok, d_model = x.shape
    e_local, _, d_out = expert_W.shape
    n_exp = N_DEV * e_local
    chunk = n_tok // N_DEV

    my_out = lax.axis_index("i")
    eids = jnp.arange(n_exp, dtype=jnp.int32)
    mask = ((route_idx[:, 0:1] == eids[None, :])
            | (route_idx[:, 1:2] == eids[None, :])).astype(jnp.float32)
    mask_pad = jnp.pad(mask, ((0, 0), (0, d_model - n_exp)))
    sel = (eids[:, None]
           == my_out * e_local + jnp.arange(e_local)[None, :]).astype(jnp.float32)
    sel_pad = jnp.pad(sel, ((0, 0), (0, d_model - e_local)))
    ew_b = expert_W.astype(jnp.bfloat16).reshape(e_local * d_model, d_out)
    ew_pack = lax.bitcast_convert_type(
        jnp.stack([ew_b[:, :d_out // 2], ew_b[:, d_out // 2:]], axis=-1),
        jnp.float32)
    packed = jnp.concatenate([x, router_W.T, mask_pad, sel_pad, ew_pack],
                             axis=0)

    ew_row0 = n_tok + n_exp + n_tok + n_exp

    def body(pk_ref, out_ref,
             gate_ref, send_ref, recv_ref, send_sems, recv_sems):
        my = lax.axis_index("i")

        barrier_sem = pltpu.get_barrier_semaphore()
        for p in range(1, N_DEV):
            pl.semaphore_signal(
                barrier_sem, inc=1,
                device_id=(lax.rem(my + p, N_DEV),),
                device_id_type=pl.DeviceIdType.MESH,
            )

        xf = pk_ref[pl.ds(0, n_tok), :]
        rwT = pk_ref[pl.ds(n_tok, n_exp), :]
        msk = pk_ref[pl.ds(n_tok + n_exp, n_tok), 0:n_exp]
        sel = pk_ref[pl.ds(n_tok + n_exp + n_tok, n_exp), 0:e_local]
        scores = lax.dot_general(
            xf, rwT, (((1,), (1,)), ((), ())),
            preferred_element_type=jnp.float32,
            precision=lax.Precision.HIGHEST)
        m = jnp.max(scores, axis=1, keepdims=True)
        gm = jnp.exp(scores - m) * msk
        denom = jnp.sum(gm, axis=1, keepdims=True)
        gate_ref[:, :] = lax.dot_general(
            gm, sel, (((1,), (0,)), ((), ())),
            preferred_element_type=jnp.float32) / denom

        w = lax.bitcast_convert_type(
            pk_ref[pl.ds(ew_row0, e_local * d_model), :], jnp.uint32)
        ew = jnp.concatenate(
            [lax.bitcast_convert_type(w << 16, jnp.float32
                                      ).astype(jnp.bfloat16),
             lax.bitcast_convert_type(w & jnp.uint32(0xFFFF0000), jnp.float32
                                      ).astype(jnp.bfloat16)],
            axis=1)

        def chunk_partial(c_start):
            x_c = pk_ref[pl.ds(c_start, chunk), :]
            xg = jnp.concatenate(
                [(gate_ref[pl.ds(c_start, chunk), k:k + 1] * x_c
                  ).astype(jnp.bfloat16) for k in range(e_local)],
                axis=1)
            return jnp.dot(xg, ew,
                           preferred_element_type=jnp.float32)

        pl.semaphore_wait(barrier_sem, N_DEV - 1)

        rdmas = {}
        for i, p in enumerate(PEER_ORDER):
            r = lax.rem(my + p, N_DEV)
            send_ref[i] = chunk_partial(r * chunk).astype(jnp.bfloat16)
            rdma = pltpu.make_async_remote_copy(
                src_ref=send_ref.at[i],
                dst_ref=recv_ref.at[N_DEV - 1 - p],
                send_sem=send_sems.at[i],
                recv_sem=recv_sems.at[N_DEV - 1 - p],
                device_id=(r,),
                device_id_type=pl.DeviceIdType.MESH,
            )
            rdma.start()
            rdmas[p] = rdma

        out = chunk_partial(my * chunk)
        for p in PEER_ORDER:
            rdmas[p].wait_recv()
            out = out + recv_ref[N_DEV - 1 - p].astype(jnp.float32)
        out_ref[:, :] = out.astype(jnp.bfloat16)

        for p in PEER_ORDER:
            rdmas[p].wait_send()

    return pl.pallas_call(
        body,
        out_shape=jax.ShapeDtypeStruct((chunk, d_out), jnp.bfloat16),
        in_specs=[
            pl.BlockSpec(memory_space=pltpu.VMEM),
        ],
        out_specs=pl.BlockSpec(memory_space=pltpu.VMEM),
        scratch_shapes=[
            pltpu.VMEM((n_tok, e_local), jnp.float32),
            pltpu.VMEM((N_DEV - 1, chunk, d_out), jnp.bfloat16),
            pltpu.VMEM((N_DEV - 1, chunk, d_out), jnp.bfloat16),
            pltpu.SemaphoreType.DMA((N_DEV - 1,)),
            pltpu.SemaphoreType.DMA((N_DEV - 1,)),
        ],
        compiler_params=pltpu.CompilerParams(collective_id=0),
    )(packed)


# device time: 13212 ns/iter; 1.1773x vs baseline; 1.1773x over previous
import jax
import jax.numpy as jnp
from jax import lax
from jax.experimental import pallas as pl
from jax.experimental.pallas import tpu as pltpu

N_DEV = 4
PEER_ORDER = (1, 3, 2)


def kernel(x, router_W, route_idx, expert_W):
    n_tok, d_model = x.shape
    e_local, _, d_out = expert_W.shape
    n_exp = N_DEV * e_local
    chunk = n_tok // N_DEV

    my_out = lax.axis_index("i")
    eids = jnp.arange(n_exp, dtype=jnp.int32)
    mask = ((route_idx[:, 0:1] == eids[None, :])
            | (route_idx[:, 1:2] == eids[None, :])).astype(jnp.float32)
    mask_pad = jnp.pad(mask, ((0, 0), (0, d_model - n_exp)))
    sel = (eids[:, None]
           == my_out * e_local + jnp.arange(e_local)[None, :]).astype(jnp.float32)
    sel_pad = jnp.pad(sel, ((0, 0), (0, d_model - e_local)))
    packed = jnp.concatenate([x, router_W.T, mask_pad, sel_pad],
                             axis=0)
    ew2 = expert_W.astype(jnp.bfloat16).reshape(e_local * d_model, d_out)

    def body(pk_ref, ew_ref, out_ref,
             gate_ref, send_ref, recv_ref, send_sems, recv_sems):
        my = lax.axis_index("i")

        barrier_sem = pltpu.get_barrier_semaphore()
        for p in range(1, N_DEV):
            pl.semaphore_signal(
                barrier_sem, inc=1,
                device_id=(lax.rem(my + p, N_DEV),),
                device_id_type=pl.DeviceIdType.MESH,
            )

        xf = pk_ref[pl.ds(0, n_tok), :]
        rwT = pk_ref[pl.ds(n_tok, n_exp), :]
        msk = pk_ref[pl.ds(n_tok + n_exp, n_tok), 0:n_exp]
        sel = pk_ref[pl.ds(n_tok + n_exp + n_tok, n_exp), 0:e_local]
        scores = lax.dot_general(
            xf, rwT, (((1,), (1,)), ((), ())),
            preferred_element_type=jnp.float32,
            precision=lax.Precision.HIGHEST)
        m = jnp.max(scores, axis=1, keepdims=True)
        gm = jnp.exp(scores - m) * msk
        denom = jnp.sum(gm, axis=1, keepdims=True)
        gate_ref[:, :] = lax.dot_general(
            gm, sel, (((1,), (0,)), ((), ())),
            preferred_element_type=jnp.float32) / denom

        def chunk_partial(c_start):
            x_c = pk_ref[pl.ds(c_start, chunk), :]
            xg = jnp.concatenate(
                [(gate_ref[pl.ds(c_start, chunk), k:k + 1] * x_c
                  ).astype(jnp.bfloat16) for k in range(e_local)],
                axis=1)
            return jnp.dot(xg, ew_ref[:, :],
                           preferred_element_type=jnp.float32)

        pl.semaphore_wait(barrier_sem, N_DEV - 1)

        rdmas = {}
        for i, p in enumerate(PEER_ORDER):
            r = lax.rem(my + p, N_DEV)
            send_ref[i] = chunk_partial(r * chunk).astype(jnp.bfloat16)
            rdma = pltpu.make_async_remote_copy(
                src_ref=send_ref.at[i],
                dst_ref=recv_ref.at[N_DEV - 1 - p],
                send_sem=send_sems.at[i],
                recv_sem=recv_sems.at[N_DEV - 1 - p],
                device_id=(r,),
                device_id_type=pl.DeviceIdType.MESH,
            )
            rdma.start()
            rdmas[p] = rdma

        out = chunk_partial(my * chunk)
        for p in PEER_ORDER:
            rdmas[p].wait_recv()
            out = out + recv_ref[N_DEV - 1 - p].astype(jnp.float32)
        out_ref[:, :] = out.astype(jnp.bfloat16)

        for p in PEER_ORDER:
            rdmas[p].wait_send()

    return pl.pallas_call(
        body,
        out_shape=jax.ShapeDtypeStruct((chunk, d_out), jnp.bfloat16),
        in_specs=[
            pl.BlockSpec(memory_space=pltpu.VMEM),
            pl.BlockSpec(memory_space=pltpu.VMEM),
        ],
        out_specs=pl.BlockSpec(memory_space=pltpu.VMEM),
        scratch_shapes=[
            pltpu.VMEM((n_tok, e_local), jnp.float32),
            pltpu.VMEM((N_DEV - 1, chunk, d_out), jnp.bfloat16),
            pltpu.VMEM((N_DEV - 1, chunk, d_out), jnp.bfloat16),
            pltpu.SemaphoreType.DMA((N_DEV - 1,)),
            pltpu.SemaphoreType.DMA((N_DEV - 1,)),
        ],
        compiler_params=pltpu.CompilerParams(collective_id=0),
    )(packed, ew2)


# device time: 12425 ns/iter; 1.2519x vs baseline; 1.0633x over previous
import jax
import jax.numpy as jnp
from jax import lax
from jax.experimental import pallas as pl
from jax.experimental.pallas import tpu as pltpu

N_DEV = 4
PEER_ORDER = (2, 3, 1)


def kernel(x, router_W, route_idx, expert_W):
    n_tok, d_model = x.shape
    e_local, _, d_out = expert_W.shape
    n_exp = N_DEV * e_local
    chunk = n_tok // N_DEV

    my_out = lax.axis_index("i")
    eids = jnp.arange(n_exp, dtype=jnp.int32)
    mask = ((route_idx[:, 0:1] == eids[None, :])
            | (route_idx[:, 1:2] == eids[None, :])).astype(jnp.float32)
    mask_pad = jnp.pad(mask, ((0, 0), (0, d_model - n_exp)))
    sel = (eids[:, None]
           == my_out * e_local + jnp.arange(e_local)[None, :]).astype(jnp.float32)
    sel_pad = jnp.pad(sel, ((0, 0), (0, d_model - e_local)))
    packed = jnp.concatenate([x, router_W.T, mask_pad, sel_pad],
                             axis=0)
    ew2 = expert_W.astype(jnp.bfloat16).reshape(e_local * d_model, d_out)

    def body(pk_ref, ew_ref, out_ref,
             gate_ref, send_ref, recv_ref, send_sems, recv_sems):
        my = lax.axis_index("i")

        barrier_sem = pltpu.get_barrier_semaphore()
        for p in range(1, N_DEV):
            pl.semaphore_signal(
                barrier_sem, inc=1,
                device_id=(lax.rem(my + p, N_DEV),),
                device_id_type=pl.DeviceIdType.MESH,
            )

        xf = pk_ref[pl.ds(0, n_tok), :]
        rwT = pk_ref[pl.ds(n_tok, n_exp), :]
        msk = pk_ref[pl.ds(n_tok + n_exp, n_tok), 0:n_exp]
        sel = pk_ref[pl.ds(n_tok + n_exp + n_tok, n_exp), 0:e_local]
        scores = lax.dot_general(
            xf, rwT, (((1,), (1,)), ((), ())),
            preferred_element_type=jnp.float32,
            precision=lax.Precision.HIGHEST)
        m = jnp.max(scores, axis=1, keepdims=True)
        gm = jnp.exp(scores - m) * msk
        denom = jnp.sum(gm, axis=1, keepdims=True)
        gate_ref[:, :] = lax.dot_general(
            gm, sel, (((1,), (0,)), ((), ())),
            preferred_element_type=jnp.float32) / denom

        def chunk_partial(c_start):
            x_c = pk_ref[pl.ds(c_start, chunk), :]
            xg = jnp.concatenate(
                [(gate_ref[pl.ds(c_start, chunk), k:k + 1] * x_c
                  ).astype(jnp.bfloat16) for k in range(e_local)],
                axis=1)
            return jnp.dot(xg, ew_ref[:, :],
                           preferred_element_type=jnp.float32)

        pl.semaphore_wait(barrier_sem, N_DEV - 1)

        rdmas = {}
        for i, p in enumerate(PEER_ORDER):
            r = lax.rem(my + p, N_DEV)
            send_ref[i] = chunk_partial(r * chunk).astype(jnp.bfloat16)
            rdma = pltpu.make_async_remote_copy(
                src_ref=send_ref.at[i],
                dst_ref=recv_ref.at[N_DEV - 1 - p],
                send_sem=send_sems.at[i],
                recv_sem=recv_sems.at[N_DEV - 1 - p],
                device_id=(r,),
                device_id_type=pl.DeviceIdType.MESH,
            )
            rdma.start()
            rdmas[p] = rdma

        out = chunk_partial(my * chunk)
        for p in PEER_ORDER:
            rdmas[p].wait_recv()
            out = out + recv_ref[N_DEV - 1 - p].astype(jnp.float32)
        out_ref[:, :] = out.astype(jnp.bfloat16)

        for p in PEER_ORDER:
            rdmas[p].wait_send()

    return pl.pallas_call(
        body,
        out_shape=jax.ShapeDtypeStruct((chunk, d_out), jnp.bfloat16),
        in_specs=[
            pl.BlockSpec(memory_space=pltpu.VMEM),
            pl.BlockSpec(memory_space=pltpu.VMEM),
        ],
        out_specs=pl.BlockSpec(memory_space=pltpu.VMEM),
        scratch_shapes=[
            pltpu.VMEM((n_tok, e_local), jnp.float32),
            pltpu.VMEM((N_DEV - 1, chunk, d_out), jnp.bfloat16),
            pltpu.VMEM((N_DEV - 1, chunk, d_out), jnp.bfloat16),
            pltpu.SemaphoreType.DMA((N_DEV - 1,)),
            pltpu.SemaphoreType.DMA((N_DEV - 1,)),
        ],
        compiler_params=pltpu.CompilerParams(collective_id=0),
    )(packed, ew2)
